# baseline (speedup 1.0000x reference)
import jax
import jax.numpy as jnp
from jax import lax
from jax.experimental import pallas as pl
from jax.experimental.pallas import tpu as pltpu
from jax.experimental.pallas import tpu_sc as plsc

B = 4
HO = WO = 192
CC = 192
OUT_B = HO * WO * CC
IN_B = OUT_B // 4
TOTAL_OUT = B * OUT_B
NS = 16
NWIN = 2
WIN = OUT_B // (2 * NWIN)
SHARE = IN_B // NS
WSHARE = WIN // NS
CHUNK = 4608
NCHUNK = SHARE // CHUNK
GROUPS = CHUNK // 16
PK = 2 * CHUNK


def _scatter_body(pkd_hbm, z_hbm, out_hbm, win_sh, pk, off_v, val_v,
                  ssem, zsem):
    c = lax.axis_index("c")
    s = lax.axis_index("s")

    def load_src(b, ch):
        return pkd_hbm.at[pl.ds(2 * (b * IN_B + s * SHARE) + ch * PK, PK)]

    def compute_chunk():
        def body(g, carry):
            iv = pk[pl.ds(g * 16, 16)]
            uv = lax.bitcast_convert_type(pk[pl.ds(CHUNK + g * 16, 16)],
                                          jnp.float32)
            rel = iv - compute_chunk.wbase
            inm = (iv >= compute_chunk.wbase) & (rel < WIN)
            off_v[pl.ds(g * 16, 16)] = jnp.where(inm, rel, iv & 0xFFFF)
            val_v[pl.ds(g * 16, 16)] = jnp.where(
                inm, uv, jnp.zeros((16,), jnp.float32))
            return carry

        lax.fori_loop(0, GROUPS, body, 0)

    for b in range(B):
        for w in range(NWIN):
            wbase = c * (NWIN * WIN) + w * WIN
            compute_chunk.wbase = wbase

            zdma = pltpu.async_copy(
                z_hbm.at[pl.ds(s * WSHARE, WSHARE)],
                win_sh.at[pl.ds(s * WSHARE, WSHARE)], zsem)
            pltpu.sync_copy(load_src(b, 0), pk)
            compute_chunk()
            zdma.wait()
            plsc.subcore_barrier()

            def chunk_step(i, carry):
                sdma = pltpu.async_copy(val_v, win_sh.at[off_v], ssem,
                                        add=True)

                @pl.when(i + 1 < NCHUNK)
                def _():
                    pltpu.sync_copy(load_src(b, i + 1), pk)

                sdma.wait()

                @pl.when(i + 1 < NCHUNK)
                def _():
                    compute_chunk()

                return carry

            lax.fori_loop(0, NCHUNK, chunk_step, 0)
            plsc.subcore_barrier()

            out_base = b * OUT_B + wbase + s * WSHARE
            pltpu.sync_copy(win_sh.at[pl.ds(s * WSHARE, WSHARE)],
                            out_hbm.at[pl.ds(out_base, WSHARE)])
            plsc.subcore_barrier()


def kernel(updates, mask):
    idx = mask.reshape(-1).astype(jnp.int32)
    upd = jax.lax.bitcast_convert_type(updates.reshape(-1), jnp.int32)
    packed = jnp.stack(
        [idx.reshape(-1, CHUNK), upd.reshape(-1, CHUNK)], axis=1).reshape(-1)
    zeros = jnp.zeros((WIN,), jnp.float32)
    mesh = plsc.VectorSubcoreMesh(core_axis_name="c", subcore_axis_name="s")
    run = pl.kernel(
        _scatter_body,
        mesh=mesh,
        out_type=jax.ShapeDtypeStruct((TOTAL_OUT,), jnp.float32),
        scratch_types=[
            pltpu.VMEM_SHARED((WIN,), jnp.float32),
            pltpu.VMEM((PK,), jnp.int32),
            pltpu.VMEM((CHUNK,), jnp.int32),
            pltpu.VMEM((CHUNK,), jnp.float32),
            pltpu.SemaphoreType.DMA,
            pltpu.SemaphoreType.DMA,
        ],
    )
    out = run(packed, zeros)
    return out.reshape(B, HO, WO, CC)
